# baseline (speedup 1.0000x reference)
import dataclasses

import jax
import jax.numpy as jnp
from jax.experimental import pallas as pl
from jax.experimental.pallas import tpu as pltpu
from jax.experimental.pallas import tpu_sc as plsc

_B, _L, _D = 16384, 200, 128
_N = _B * _L
_HOURS = 72
_HSTRIDE = 64
_W = 128


def _combo_body(minute_ref, hour_ref, out_ref):
    out_ref[...] = minute_ref[...] + hour_ref[0]


def _build_combo(minute_pad, hour3):
    return pl.pallas_call(
        _combo_body,
        grid=(_HOURS,),
        in_specs=[
            pl.BlockSpec((_HSTRIDE, _D), lambda h: (0, 0)),
            pl.BlockSpec((1, 1, _D), lambda h: (h, 0, 0)),
        ],
        out_specs=pl.BlockSpec((_HSTRIDE, _D), lambda h: (h, 0)),
        out_shape=jax.ShapeDtypeStruct((_HOURS * _HSTRIDE, _D), jnp.float32),
    )(minute_pad, hour3)


_WOUT = 256


def _sc_gather(combo, xi):
    mesh = plsc.VectorSubcoreMesh(
        core_axis_name="core", subcore_axis_name="subcore"
    )
    nrows = _HOURS * _HSTRIDE
    rows_per_sub = nrows // 16
    cp = pltpu.CompilerParams()
    if "needs_layout_passes" in pltpu.CompilerParams.__dataclass_fields__:
        cp = dataclasses.replace(cp, needs_layout_passes=False)

    @pl.kernel(
        compiler_params=cp,
        out_type=jax.ShapeDtypeStruct((_N, _D), jnp.float32),
        mesh=mesh,
        scratch_types=[
            pltpu.VMEM((_W,), jnp.int32),
            pltpu.VMEM((_W,), jnp.int32),
            pltpu.VMEM_SHARED((nrows, _D), jnp.float32),
        ],
    )
    def k(combo_hbm, xi_hbm, out_hbm, idx_a, idx_b, combo_sh):
        sid = jax.lax.axis_index("subcore")
        sl = pl.ds(sid * rows_per_sub, rows_per_sub)
        pltpu.sync_copy(combo_hbm.at[sl], combo_sh.at[sl])
        plsc.subcore_barrier()

        io = jax.lax.iota(jnp.int32, 16)

        def body(x_v, o_v):
            xr = x_v.at[0]
            for half, idx_ref in ((0, idx_a), (1, idx_b)):
                for i in range(_W // 16):
                    base = 2 * (half * _W + i * 16)
                    h = plsc.load_gather(xr, [base + 2 * io])
                    m = plsc.load_gather(xr, [base + 2 * io + 1])
                    idx_ref[pl.ds(i * 16, 16)] = h * _HSTRIDE + m
                pltpu.sync_copy(
                    combo_sh.at[idx_ref], o_v.at[pl.ds(half * _W, _W)]
                )

        pltpu.emit_pipeline(
            body,
            grid=(_N // _WOUT,),
            in_specs=[pl.BlockSpec((1, 2 * _WOUT), lambda i: (0, i))],
            out_specs=[pl.BlockSpec((_WOUT, _D), lambda i: (i, 0))],
            core_axis_name=("core", "subcore"),
            dimension_semantics=(pltpu.PARALLEL,),
        )(xi_hbm, out_hbm)

    return k(combo, xi)


def kernel(x, minute_table, hour_table):
    x = x.astype(jnp.int32)
    minute_pad = jnp.pad(minute_table, ((0, _HSTRIDE - 60), (0, 0)))
    hour3 = hour_table.reshape(_HOURS, 1, _D)
    combo = _build_combo(minute_pad, hour3)
    xi = x.reshape(1, 2 * _N)
    out = _sc_gather(combo, xi)
    return out.reshape(_B, _L, _D)


# speedup vs baseline: 5.5046x; 5.5046x over previous
import dataclasses

import jax
import jax.numpy as jnp
from jax.experimental import pallas as pl
from jax.experimental.pallas import tpu as pltpu
from jax.experimental.pallas import tpu_sc as plsc

_B, _L, _D = 16384, 200, 128
_N = _B * _L
_HOURS = 72
_HSTRIDE = 64
_W = 128


def _combo_body(minute_ref, hour_ref, out_ref):
    out_ref[...] = minute_ref[...] + hour_ref[0]


def _build_combo(minute_pad, hour3):
    return pl.pallas_call(
        _combo_body,
        grid=(_HOURS,),
        in_specs=[
            pl.BlockSpec((_HSTRIDE, _D), lambda h: (0, 0)),
            pl.BlockSpec((1, 1, _D), lambda h: (h, 0, 0)),
        ],
        out_specs=pl.BlockSpec((_HSTRIDE, _D), lambda h: (h, 0)),
        out_shape=jax.ShapeDtypeStruct((_HOURS * _HSTRIDE, _D), jnp.float32),
    )(minute_pad, hour3)


_WOUT = 256


def _sc_gather(combo, x0, x1):
    mesh = plsc.VectorSubcoreMesh(
        core_axis_name="core", subcore_axis_name="subcore"
    )
    nrows = _HOURS * _HSTRIDE
    rows_per_sub = nrows // 16

    @pl.kernel(
        out_type=jax.ShapeDtypeStruct((_N, _D), jnp.float32),
        mesh=mesh,
        scratch_types=[
            pltpu.VMEM((_W,), jnp.int32),
            pltpu.VMEM((_W,), jnp.int32),
            pltpu.VMEM_SHARED((nrows, _D), jnp.float32),
        ],
    )
    def k(combo_hbm, x0_hbm, x1_hbm, out_hbm, idx_a, idx_b, combo_sh):
        sid = jax.lax.axis_index("subcore")
        sl = pl.ds(sid * rows_per_sub, rows_per_sub)
        pltpu.sync_copy(combo_hbm.at[sl], combo_sh.at[sl])
        plsc.subcore_barrier()

        def body(x0_v, x1_v, o_v):
            x0r = x0_v.at[0]
            x1r = x1_v.at[0]
            for half, idx_ref in ((0, idx_a), (1, idx_b)):
                for i in range(_W // 16):
                    s = pl.ds(half * _W + i * 16, 16)
                    idx_ref[pl.ds(i * 16, 16)] = x0r[s] * _HSTRIDE + x1r[s]
                pltpu.sync_copy(
                    combo_sh.at[idx_ref], o_v.at[pl.ds(half * _W, _W)]
                )

        pltpu.emit_pipeline(
            body,
            grid=(_N // _WOUT,),
            in_specs=[
                pl.BlockSpec((1, _WOUT), lambda i: (0, i)),
                pl.BlockSpec((1, _WOUT), lambda i: (0, i)),
            ],
            out_specs=[pl.BlockSpec((_WOUT, _D), lambda i: (i, 0))],
            core_axis_name=("core", "subcore"),
            dimension_semantics=(pltpu.PARALLEL,),
        )(x0_hbm, x1_hbm, out_hbm)

    return k(combo, x0, x1)


def kernel(x, minute_table, hour_table):
    x = x.astype(jnp.int32)
    minute_pad = jnp.pad(minute_table, ((0, _HSTRIDE - 60), (0, 0)))
    hour3 = hour_table.reshape(_HOURS, 1, _D)
    combo = _build_combo(minute_pad, hour3)
    x0 = x[:, :, 0].reshape(1, _N)
    x1 = x[:, :, 1].reshape(1, _N)
    out = _sc_gather(combo, x0, x1)
    return out.reshape(_B, _L, _D)


# speedup vs baseline: 9.3051x; 1.6904x over previous
import jax
import jax.numpy as jnp
from jax.experimental import pallas as pl
from jax.experimental.pallas import tpu as pltpu
from jax.experimental.pallas import tpu_sc as plsc

_B, _L, _D = 16384, 200, 128
_N = _B * _L
_HOURS = 72
_HSTRIDE = 64
_W = 128
_NROWS = _HOURS * _HSTRIDE
_RPS = _NROWS // 16


def _sc_gather(minute_pad, hour_table, x0, x1):
    mesh = plsc.VectorSubcoreMesh(
        core_axis_name="core", subcore_axis_name="subcore"
    )

    @pl.kernel(
        out_type=jax.ShapeDtypeStruct((_N, _D), jnp.float32),
        mesh=mesh,
        scratch_types=[
            pltpu.VMEM((_W,), jnp.int32),
            pltpu.VMEM((_HSTRIDE, _D), jnp.float32),
            pltpu.VMEM((_HOURS, _D), jnp.float32),
            pltpu.VMEM((_RPS, _D), jnp.float32),
            pltpu.VMEM_SHARED((_NROWS, _D), jnp.float32),
        ],
    )
    def k(min_hbm, hour_hbm, x0_hbm, x1_hbm, out_hbm,
          idx_ref, min_v, hour_v, cbuf, combo_sh):
        sid = jax.lax.axis_index("subcore")
        pltpu.sync_copy(min_hbm, min_v.at[pl.ds(0, 60)])
        pltpu.sync_copy(hour_hbm, hour_v)
        base = sid * _RPS

        @pl.loop(0, _RPS)
        def _(r):
            row = base + r
            h = jax.lax.shift_right_logical(row, 6)
            m = jax.lax.bitwise_and(row, _HSTRIDE - 1)
            for i in range(_D // 16):
                s = pl.ds(i * 16, 16)
                cbuf[r, s] = hour_v[h, s] + min_v[m, s]

        sl = pl.ds(base, _RPS)
        pltpu.sync_copy(cbuf, combo_sh.at[sl])
        plsc.subcore_barrier()

        def body(x0_v, x1_v, o_v):
            x0r = x0_v.at[0]
            x1r = x1_v.at[0]
            for i in range(_W // 16):
                s = pl.ds(i * 16, 16)
                idx_ref[s] = x0r[s] * _HSTRIDE + x1r[s]
            pltpu.sync_copy(combo_sh.at[idx_ref], o_v)

        pltpu.emit_pipeline(
            body,
            grid=(_N // _W,),
            in_specs=[
                pl.BlockSpec((1, _W), lambda i: (0, i)),
                pl.BlockSpec((1, _W), lambda i: (0, i)),
            ],
            out_specs=[pl.BlockSpec((_W, _D), lambda i: (i, 0))],
            core_axis_name=("core", "subcore"),
            dimension_semantics=(pltpu.PARALLEL,),
        )(x0_hbm, x1_hbm, out_hbm)

    return k(minute_pad, hour_table, x0, x1)


def kernel(x, minute_table, hour_table):
    x = x.astype(jnp.int32)
    x0 = x[:, :, 0].reshape(1, _N)
    x1 = x[:, :, 1].reshape(1, _N)
    out = _sc_gather(minute_table, hour_table, x0, x1)
    return out.reshape(_B, _L, _D)


# speedup vs baseline: 9.3121x; 1.0008x over previous
import jax
import jax.numpy as jnp
from jax.experimental import pallas as pl
from jax.experimental.pallas import tpu as pltpu
from jax.experimental.pallas import tpu_sc as plsc

_B, _L, _D = 16384, 200, 128
_N = _B * _L
_HOURS = 72
_HSTRIDE = 64
_W = 128
_NROWS = _HOURS * _HSTRIDE
_RPS = _NROWS // 16


def _sc_gather(minute_pad, hour_table, x0, x1):
    mesh = plsc.VectorSubcoreMesh(
        core_axis_name="core", subcore_axis_name="subcore"
    )

    @pl.kernel(
        out_type=jax.ShapeDtypeStruct((_N, _D), jnp.float32),
        mesh=mesh,
        scratch_types=[
            pltpu.VMEM((_W,), jnp.int32),
            pltpu.VMEM((_HSTRIDE, _D), jnp.float32),
            pltpu.VMEM((_HOURS, _D), jnp.float32),
            pltpu.VMEM((_RPS, _D), jnp.float32),
            pltpu.VMEM_SHARED((_NROWS, _D), jnp.float32),
        ],
    )
    def k(min_hbm, hour_hbm, x0_hbm, x1_hbm, out_hbm,
          idx_ref, min_v, hour_v, cbuf, combo_sh):
        sid = jax.lax.axis_index("subcore")
        pltpu.sync_copy(min_hbm, min_v.at[pl.ds(0, 60)])
        pltpu.sync_copy(hour_hbm, hour_v)
        base = sid * _RPS

        @pl.loop(0, _RPS)
        def _(r):
            row = base + r
            h = jax.lax.shift_right_logical(row, 6)
            m = jax.lax.bitwise_and(row, _HSTRIDE - 1)
            for i in range(_D // 16):
                s = pl.ds(i * 16, 16)
                cbuf[r, s] = hour_v[h, s] + min_v[m, s]

        sl = pl.ds(base, _RPS)
        pltpu.sync_copy(cbuf, combo_sh.at[sl])
        plsc.subcore_barrier()

        def body(x0_v, x1_v, o_v):
            x0r = x0_v.at[0]
            x1r = x1_v.at[0]
            for i in range(_W // 16):
                s = pl.ds(i * 16, 16)
                idx_ref[s] = x0r[s] * _HSTRIDE + x1r[s]
            pltpu.sync_copy(combo_sh.at[idx_ref], o_v)

        pltpu.emit_pipeline(
            body,
            grid=(_N // _W,),
            in_specs=[
                pl.BlockSpec(
                    (1, _W), lambda i: (0, i),
                    pipeline_mode=pl.Buffered(buffer_count=4),
                ),
                pl.BlockSpec(
                    (1, _W), lambda i: (0, i),
                    pipeline_mode=pl.Buffered(buffer_count=4),
                ),
            ],
            out_specs=[pl.BlockSpec((_W, _D), lambda i: (i, 0))],
            core_axis_name=("core", "subcore"),
            dimension_semantics=(pltpu.PARALLEL,),
        )(x0_hbm, x1_hbm, out_hbm)

    return k(minute_pad, hour_table, x0, x1)


def kernel(x, minute_table, hour_table):
    x = x.astype(jnp.int32)
    x0 = x[:, :, 0].reshape(1, _N)
    x1 = x[:, :, 1].reshape(1, _N)
    out = _sc_gather(minute_table, hour_table, x0, x1)
    return out.reshape(_B, _L, _D)
